# Initial kernel scaffold; baseline (speedup 1.0000x reference)
import jax
import jax.numpy as jnp
from jax.experimental import pallas as pl


def kernel(token_ids, token_table):
    raise NotImplementedError("write your pallas kernel here")


# speedup vs baseline: 1.7794x; 1.7794x over previous
import functools
import math

import jax
import jax.numpy as jnp
import numpy as np
from jax import lax
from jax.experimental import pallas as pl
from jax.experimental.pallas import tpu as pltpu
from jax.experimental.pallas import tpu_sc as plsc

VOCAB = 100000
D_MODEL = 1024
BATCH = 4
SEQ = 2048

NC = 2
NS = 16
NW = NC * NS
LANES = 16

POS_PER_W = SEQ // NW
R = 32
H = POS_PER_W // R
GROUPS_PER_ROW = D_MODEL // LANES


def _pe_table() -> np.ndarray:
    pos = np.arange(SEQ, dtype=np.float64)[:, None]
    i = np.arange(0, D_MODEL, 2, dtype=np.float64)
    div = np.exp(-math.log(10000.0) * i / D_MODEL)
    pe = np.zeros((SEQ, D_MODEL), dtype=np.float64)
    pe[:, 0::2] = np.sin(pos * div)
    pe[:, 1::2] = np.cos(pos * div)
    return pe.astype(np.float32)


_PE = _pe_table()


@functools.partial(
    pl.kernel,
    out_type=jax.ShapeDtypeStruct((BATCH * SEQ, D_MODEL), jnp.float32),
    mesh=plsc.VectorSubcoreMesh(
        core_axis_name="c", subcore_axis_name="s", num_cores=NC,
        num_subcores=NS),
    scratch_types=[
        pltpu.VMEM((BATCH * POS_PER_W,), jnp.int32),
        pltpu.VMEM((R, D_MODEL), jnp.float32),
        pltpu.VMEM((R, D_MODEL), jnp.float32),
        pltpu.VMEM((R, D_MODEL), jnp.float32),
        pltpu.SemaphoreType.DMA,
        pltpu.SemaphoreType.DMA,
    ],
)
def _embed_sc(ids_hbm, table_hbm, pe_hbm, out_hbm,
              idx_v, pe_v, buf0, buf1, gsem0, gsem1):
    wid = lax.axis_index("s") * NC + lax.axis_index("c")
    pos_base = wid * POS_PER_W

    for b in range(BATCH):
        pltpu.sync_copy(
            ids_hbm.at[pl.ds(b * SEQ + pos_base, POS_PER_W)],
            idx_v.at[pl.ds(b * POS_PER_W, POS_PER_W)],
        )

    bufs = (buf0, buf1)
    sems = (gsem0, gsem1)
    chunks = [(h, b) for h in range(H) for b in range(BATCH)]

    def add_pe(buf):
        def row_body(r, carry):
            for c in range(GROUPS_PER_ROW):
                plsc.addupdate(
                    buf.at[r, pl.ds(c * LANES, LANES)],
                    pe_v[r, pl.ds(c * LANES, LANES)],
                )
            return carry
        lax.fori_loop(0, R, row_body, 0)

    def issue_gather(k):
        h, b = chunks[k]
        return pltpu.async_copy(
            table_hbm.at[idx_v.at[pl.ds(b * POS_PER_W + h * R, R)]],
            bufs[k % 2],
            sems[k % 2],
        )

    copies = {0: issue_gather(0)}
    for k, (h, b) in enumerate(chunks):
        if b == 0:
            pltpu.sync_copy(pe_hbm.at[pl.ds(pos_base + h * R, R)], pe_v)
        if k + 1 < len(chunks):
            copies[k + 1] = issue_gather(k + 1)
        copies.pop(k).wait()
        buf = bufs[k % 2]
        add_pe(buf)
        pltpu.sync_copy(buf, out_hbm.at[pl.ds(b * SEQ + pos_base + h * R, R)])


def kernel(token_ids, token_table):
    ids = token_ids.reshape(-1).astype(jnp.int32)
    pe = jnp.asarray(_PE)
    out = _embed_sc(ids, token_table, pe)
    return out.reshape(BATCH, SEQ, D_MODEL)


# speedup vs baseline: 2.0754x; 1.1663x over previous
import functools
import math

import jax
import jax.numpy as jnp
import numpy as np
from jax import lax
from jax.experimental import pallas as pl
from jax.experimental.pallas import tpu as pltpu
from jax.experimental.pallas import tpu_sc as plsc

VOCAB = 100000
D_MODEL = 1024
BATCH = 4
SEQ = 2048

NC = 2
NS = 16
NW = NC * NS
LANES = 16

POS_PER_W = SEQ // NW
R = 16
H = POS_PER_W // R
NCHUNK = H * BATCH
NBUF = 5
DEPTH = 3
GROUPS_PER_ROW = D_MODEL // LANES


def _pe_table() -> np.ndarray:
    pos = np.arange(SEQ, dtype=np.float64)[:, None]
    i = np.arange(0, D_MODEL, 2, dtype=np.float64)
    div = np.exp(-math.log(10000.0) * i / D_MODEL)
    pe = np.zeros((SEQ, D_MODEL), dtype=np.float64)
    pe[:, 0::2] = np.sin(pos * div)
    pe[:, 1::2] = np.cos(pos * div)
    return pe.astype(np.float32)


_PE = _pe_table()


@functools.partial(
    pl.kernel,
    out_type=jax.ShapeDtypeStruct((BATCH * SEQ, D_MODEL), jnp.float32),
    mesh=plsc.VectorSubcoreMesh(
        core_axis_name="c", subcore_axis_name="s", num_cores=NC,
        num_subcores=NS),
    scratch_types=[
        pltpu.VMEM((BATCH * POS_PER_W,), jnp.int32),
        [pltpu.VMEM((R, D_MODEL), jnp.float32)] * 2,
        [pltpu.VMEM((R, D_MODEL), jnp.float32)] * NBUF,
        [pltpu.SemaphoreType.DMA] * 2,
        [pltpu.SemaphoreType.DMA] * NBUF,
        [pltpu.SemaphoreType.DMA] * NBUF,
    ],
)
def _embed_sc(ids_hbm, table_hbm, pe_hbm, out_hbm,
              idx_v, pe_bufs, bufs, pe_sems, gsems, ssems):
    wid = lax.axis_index("s") * NC + lax.axis_index("c")
    pos_base = wid * POS_PER_W

    for b in range(BATCH):
        pltpu.sync_copy(
            ids_hbm.at[pl.ds(b * SEQ + pos_base, POS_PER_W)],
            idx_v.at[pl.ds(b * POS_PER_W, POS_PER_W)],
        )

    def add_pe(buf, pe_v):
        def row_body(r, carry):
            for c in range(GROUPS_PER_ROW):
                plsc.addupdate(
                    buf.at[r, pl.ds(c * LANES, LANES)],
                    pe_v[r, pl.ds(c * LANES, LANES)],
                )
            return carry
        lax.fori_loop(0, R, row_body, 0)

    def issue_gather(k):
        h, b = divmod(k, BATCH)
        return pltpu.async_copy(
            table_hbm.at[idx_v.at[pl.ds(b * POS_PER_W + h * R, R)]],
            bufs[k % NBUF],
            gsems[k % NBUF],
        )

    def issue_pe(h):
        return pltpu.async_copy(
            pe_hbm.at[pl.ds(pos_base + h * R, R)],
            pe_bufs[h % 2],
            pe_sems[h % 2],
        )

    pe_copies = {0: issue_pe(0)}
    gathers = {k: issue_gather(k) for k in range(DEPTH)}
    stores = {}

    for k in range(NCHUNK):
        h, b = divmod(k, BATCH)
        if b == 0:
            pe_copies.pop(h).wait()
            if h + 1 < H:
                pe_copies[h + 1] = issue_pe(h + 1)
        if k + DEPTH < NCHUNK:
            if k - (NBUF - DEPTH) >= 0:
                stores.pop(k - (NBUF - DEPTH)).wait()
            gathers[k + DEPTH] = issue_gather(k + DEPTH)
        gathers.pop(k).wait()
        buf = bufs[k % NBUF]
        add_pe(buf, pe_bufs[h % 2])
        stores[k] = pltpu.async_copy(
            buf, out_hbm.at[pl.ds(b * SEQ + pos_base + h * R, R)],
            ssems[k % NBUF],
        )
    for k in sorted(stores):
        stores.pop(k).wait()


def kernel(token_ids, token_table):
    ids = token_ids.reshape(-1).astype(jnp.int32)
    pe = jnp.asarray(_PE)
    out = _embed_sc(ids, token_table, pe)
    return out.reshape(BATCH, SEQ, D_MODEL)
